# baseline (speedup 1.0000x reference)
import functools
import math

import jax
import jax.numpy as jnp
from jax import lax
from jax.experimental import pallas as pl
from jax.experimental.pallas import tpu as pltpu
from jax.experimental.pallas import tpu_sc as plsc

D_MODEL = 64
SCALE = math.sqrt(D_MODEL)

NC = 2
NS = 16
NW = NC * NS
LANES = 16

BLK = 256
NBUF = 2

CH = 128


def _make_pairer(vocab: int):
    nchunk = vocab // CH
    rem = vocab - nchunk * CH
    cpw = nchunk // NW
    nleft = nchunk - cpw * NW
    assert cpw % NBUF == 0 and rem % 2 == 0

    mesh = plsc.VectorSubcoreMesh(
        core_axis_name="c", subcore_axis_name="s",
        num_cores=NC, num_subcores=NS,
    )

    @functools.partial(
        pl.kernel,
        mesh=mesh,
        compiler_params=pltpu.CompilerParams(needs_layout_passes=False),
        out_type=jax.ShapeDtypeStruct((vocab // 2, 2 * D_MODEL), jnp.float32),
        scratch_types=[
            [pltpu.VMEM((D_MODEL, CH), jnp.float32) for _ in range(NBUF)],
            [pltpu.VMEM((CH // 2, 2 * D_MODEL), jnp.float32) for _ in range(NBUF)],
            [pltpu.SemaphoreType.DMA for _ in range(NBUF)],
            [pltpu.SemaphoreType.DMA for _ in range(NBUF)],
        ],
    )
    def k(tabt_hbm, tail2_hbm, out_hbm, inb, outb, gsem, ssem):
        wid = lax.axis_index("s") * NC + lax.axis_index("c")
        c0 = wid * cpw
        lane = lax.iota(jnp.int32, LANES)

        def start_in(c, b):
            pltpu.async_copy(tabt_hbm.at[:, pl.ds(c * CH, CH)], inb[b], gsem[b])

        def wait_in(c, b):
            pltpu.make_async_copy(
                tabt_hbm.at[:, pl.ds(c * CH, CH)], inb[b], gsem[b]).wait()

        def out_ref(c):
            return out_hbm.at[pl.ds(c * (CH // 2), CH // 2)]

        def transpose(b):
            @plsc.parallel_loop(0, CH, 1, unroll=8)
            def col_body(v):
                j = lax.shift_right_logical(v, 1)
                base = lax.shift_left(lax.bitwise_and(v, 1), 6)
                colv = lane * 0 + v
                for c in range(D_MODEL // LANES):
                    rowv = lane + (c * LANES)
                    vals = plsc.load_gather(inb[b], [rowv, colv])
                    outb[b][j, pl.ds(base + c * LANES, LANES)] = vals

        for b in range(NBUF):
            start_in(c0 + b, b)

        def group(g, carry):
            for b in range(NBUF):
                c = c0 + g * NBUF + b
                wait_in(c, b)
                @pl.when(g > 0)
                def _():
                    pltpu.make_async_copy(
                        outb[b], out_ref(c - NBUF), ssem[b]).wait()
                transpose(b)
                pltpu.async_copy(outb[b], out_ref(c), ssem[b])
                start_in(c + NBUF, b)
            return carry

        lax.fori_loop(0, cpw // NBUF - 1, group, 0)
        for b in range(NBUF):
            c = c0 + (cpw // NBUF - 1) * NBUF + b
            wait_in(c, b)
            pltpu.make_async_copy(outb[b], out_ref(c - NBUF), ssem[b]).wait()
            transpose(b)
            pltpu.async_copy(outb[b], out_ref(c), ssem[b])
        for b in range(NBUF):
            c = c0 + (cpw // NBUF - 1) * NBUF + b
            pltpu.make_async_copy(outb[b], out_ref(c), ssem[b]).wait()

        @pl.when(wid >= NW - nleft)
        def _():
            c = NW * cpw + (wid - (NW - nleft))
            start_in(c, 0)
            wait_in(c, 0)
            transpose(0)
            pltpu.async_copy(outb[0], out_ref(c), ssem[0])
            pltpu.make_async_copy(outb[0], out_ref(c), ssem[0]).wait()

        @pl.when(wid == 0)
        def _():
            pltpu.sync_copy(tail2_hbm, outb[0].at[pl.ds(0, rem // 2)])
            pltpu.sync_copy(
                outb[0].at[pl.ds(0, rem // 2)],
                out_hbm.at[pl.ds(nchunk * CH // 2, rem // 2)])

    return k


def _make_sc_gather(T_DIM: int, B_DIM: int):
    n_tok = T_DIM * B_DIM
    tok_per_w = n_tok // NW
    blks_per_w = tok_per_w // BLK
    assert blks_per_w % NBUF == 0 and B_DIM % BLK == 0
    groups = blks_per_w // NBUF

    mesh = plsc.VectorSubcoreMesh(
        core_axis_name="c", subcore_axis_name="s",
        num_cores=NC, num_subcores=NS,
    )

    @functools.partial(
        pl.kernel,
        mesh=mesh,
        compiler_params=pltpu.CompilerParams(needs_layout_passes=False),
        out_type=jax.ShapeDtypeStruct((T_DIM, D_MODEL, B_DIM), jnp.float32),
        scratch_types=[
            pltpu.VMEM((tok_per_w,), jnp.int32),
            [pltpu.VMEM((BLK,), jnp.int32) for _ in range(NBUF)],
            [pltpu.VMEM((BLK, 2 * D_MODEL), jnp.float32) for _ in range(NBUF)],
            [pltpu.VMEM((D_MODEL, BLK), jnp.float32) for _ in range(NBUF)],
            [pltpu.SemaphoreType.DMA for _ in range(NBUF)],
            [pltpu.SemaphoreType.DMA for _ in range(NBUF)],
        ],
    )
    def k(idx_hbm, tab2_hbm, out_hbm, idx_all, pidx, rows, oblk, gsem, ssem):
        wid = lax.axis_index("s") * NC + lax.axis_index("c")
        tok0 = wid * tok_per_w
        pltpu.sync_copy(idx_hbm.at[pl.ds(tok0, tok_per_w)], idx_all)
        lane = lax.iota(jnp.int32, LANES)

        def start_gather(blk, b):
            boff = blk * BLK

            @plsc.parallel_loop(0, BLK // LANES, 1, unroll=8)
            def mk_pidx(g):
                v = idx_all[pl.ds(boff + g * LANES, LANES)]
                pidx[b][pl.ds(g * LANES, LANES)] = lax.shift_right_logical(v, 1)

            pltpu.async_copy(tab2_hbm.at[pidx[b]], rows[b], gsem[b])

        def wait_gather(b):
            pltpu.make_async_copy(tab2_hbm.at[pidx[b]], rows[b], gsem[b]).wait()

        def out_ref(blk):
            gtok = tok0 + blk * BLK
            t = gtok // B_DIM
            b0 = lax.rem(gtok, B_DIM)
            return out_hbm.at[t, :, pl.ds(b0, BLK)]

        def compute(blk, b):
            boff = blk * BLK
            for g in range(BLK // LANES):
                v = idx_all[pl.ds(boff + g * LANES, LANES)]
                col0 = lax.shift_left(lax.bitwise_and(v, 1), 6)
                rowv = lane + (g * LANES)

                @plsc.parallel_loop(0, D_MODEL, 1, unroll=16)
                def d_body(d):
                    vals = plsc.load_gather(rows[b], [rowv, col0 + d])
                    oblk[b][d, pl.ds(g * LANES, LANES)] = vals * SCALE

        for b in range(NBUF):
            start_gather(b, b)

        def group_body(grp, carry):
            for b in range(NBUF):
                blk = grp * NBUF + b
                wait_gather(b)
                @pl.when(grp > 0)
                def _():
                    pltpu.make_async_copy(
                        oblk[b], out_ref(blk - NBUF), ssem[b]).wait()
                compute(blk, b)
                pltpu.async_copy(oblk[b], out_ref(blk), ssem[b])
                start_gather(blk + NBUF, b)
            return carry

        lax.fori_loop(0, groups - 1, group_body, 0)

        for b in range(NBUF):
            blk = (groups - 1) * NBUF + b
            wait_gather(b)
            pltpu.make_async_copy(oblk[b], out_ref(blk - NBUF), ssem[b]).wait()
            compute(blk, b)
            pltpu.async_copy(oblk[b], out_ref(blk), ssem[b])
        for b in range(NBUF):
            blk = (groups - 1) * NBUF + b
            pltpu.make_async_copy(oblk[b], out_ref(blk), ssem[b]).wait()

    return k


def kernel(x, table):
    B_DIM, T_DIM = x.shape
    idx = x.T.reshape(-1).astype(jnp.int32)
    tab2 = table.reshape(table.shape[0] // 2, 2 * D_MODEL)
    out3 = _make_sc_gather(T_DIM, B_DIM)(idx, tab2)
    return out3.transpose(2, 0, 1)


# speedup vs baseline: 1.0372x; 1.0372x over previous
import functools
import math

import jax
import jax.numpy as jnp
from jax import lax
from jax.experimental import pallas as pl
from jax.experimental.pallas import tpu as pltpu
from jax.experimental.pallas import tpu_sc as plsc

D_MODEL = 64
SCALE = math.sqrt(D_MODEL)

NC = 2
NS = 16
NW = NC * NS
LANES = 16
D_VECS = D_MODEL // LANES

CHUNK = 128
NBUF = 4


def _make_sc_gather(B: int):
    assert B % (NW * CHUNK * NBUF) == 0
    b_per_w = B // NW
    steps = b_per_w // CHUNK
    groups = steps // NBUF

    mesh = plsc.VectorSubcoreMesh(
        core_axis_name="c", subcore_axis_name="s",
        num_cores=NC, num_subcores=NS,
    )

    @functools.partial(
        pl.kernel,
        mesh=mesh,
        compiler_params=pltpu.CompilerParams(use_tc_tiling_on_sc=False),
        out_type=jax.ShapeDtypeStruct((B, D_MODEL), jnp.float32),
        scratch_types=[
            pltpu.VMEM((b_per_w,), jnp.int32),
            [pltpu.VMEM((CHUNK, D_MODEL), jnp.float32) for _ in range(NBUF)],
            [pltpu.VMEM((CHUNK, D_MODEL), jnp.float32) for _ in range(NBUF)],
            [pltpu.SemaphoreType.DMA for _ in range(NBUF)],
            [pltpu.SemaphoreType.DMA for _ in range(NBUF)],
        ],
    )
    def k(idx_hbm, table_hbm, out_hbm, idx_all, rows, obuf, gsem, ssem):
        wid = lax.axis_index("s") * NC + lax.axis_index("c")
        base = wid * b_per_w
        pltpu.sync_copy(idx_hbm.at[pl.ds(base, b_per_w)], idx_all)

        def gather(s, b):
            pltpu.async_copy(
                table_hbm.at[idx_all.at[pl.ds(s * CHUNK, CHUNK)]],
                rows[b], gsem[b])

        def wait_gather(s, b):
            pltpu.make_async_copy(
                table_hbm.at[idx_all.at[pl.ds(s * CHUNK, CHUNK)]],
                rows[b], gsem[b]).wait()

        def scale(b):
            @plsc.parallel_loop(0, CHUNK, 1, unroll=8)
            def body(r):
                for c in range(D_VECS):
                    sl = pl.ds(c * LANES, LANES)
                    obuf[b][r, sl] = rows[b][r, sl] * SCALE

        def store(s, b):
            pltpu.async_copy(
                obuf[b], out_hbm.at[pl.ds(base + s * CHUNK, CHUNK)], ssem[b])

        def wait_store(s, b):
            pltpu.make_async_copy(
                obuf[b], out_hbm.at[pl.ds(base + s * CHUNK, CHUNK)],
                ssem[b]).wait()

        for b in range(NBUF):
            gather(b, b)

        def group_body(g, carry):
            for b in range(NBUF):
                s = g * NBUF + b
                wait_gather(s, b)

                @pl.when(g > 0)
                def _():
                    wait_store(s - NBUF, b)

                scale(b)
                store(s, b)
                gather(s + NBUF, b)
            return carry

        lax.fori_loop(0, groups - 1, group_body, 0)

        for b in range(NBUF):
            s = (groups - 1) * NBUF + b
            wait_gather(s, b)
            wait_store(s - NBUF, b)
            scale(b)
            store(s, b)
        for b in range(NBUF):
            s = (groups - 1) * NBUF + b
            wait_store(s, b)

    return k


def kernel(x, table):
    B, T = x.shape
    flat_idx = x.reshape(-1).astype(jnp.int32)
    out = _make_sc_gather(flat_idx.shape[0])(flat_idx, table)
    return out.reshape(B, T, D_MODEL)
